# Initial kernel scaffold; baseline (speedup 1.0000x reference)
import jax
import jax.numpy as jnp
from jax.experimental import pallas as pl


def kernel(edge_index, node_pos, group_pos, means, betas, W, b):
    raise NotImplementedError("write your pallas kernel here")


# speedup vs baseline: 1.6105x; 1.6105x over previous
import functools
import math

import jax
import jax.numpy as jnp
from jax import lax
from jax.experimental import pallas as pl
from jax.experimental.pallas import tpu as pltpu
from jax.experimental.pallas import tpu_sc as plsc

CUTOFF_LOWER = 0.0
CUTOFF_UPPER = 10.0

_NUM_CORES = 2
_NUM_SUBCORES = 16
_NW = _NUM_CORES * _NUM_SUBCORES
_LANES = 16
_CHUNK = 2000


def _sc_gather_kernel(src, dst, node_flat, group_flat):
    n_edges = src.shape[0]
    edges_per_w = n_edges // _NW
    n_chunks = edges_per_w // _CHUNK

    mesh = plsc.VectorSubcoreMesh(core_axis_name="c", subcore_axis_name="s")

    def body(src_hbm, dst_hbm, node_hbm, group_hbm, ew_hbm, ev_hbm,
             node_v, group_v, src_v, dst_v, ew_v, ev_v):
        cid = lax.axis_index("c")
        sid = lax.axis_index("s")
        wid = sid * _NUM_CORES + cid
        base_w = wid * edges_per_w
        pltpu.sync_copy(node_hbm, node_v)
        pltpu.sync_copy(group_hbm, group_v)

        lane_iota3 = lax.iota(jnp.int32, _LANES) * 3

        def chunk_body(k, carry):
            base = base_w + k * _CHUNK
            pltpu.sync_copy(src_hbm.at[pl.ds(base, _CHUNK)], src_v)
            pltpu.sync_copy(dst_hbm.at[pl.ds(base, _CHUNK)], dst_v)

            def vec_body(j, carry2):
                off = j * _LANES
                si = src_v[pl.ds(off, _LANES)] * 3
                di = dst_v[pl.ds(off, _LANES)] * 3
                dx = (plsc.load_gather(node_v, [si])
                      - plsc.load_gather(group_v, [di]))
                dy = (plsc.load_gather(node_v, [si + 1])
                      - plsc.load_gather(group_v, [di + 1]))
                dz = (plsc.load_gather(node_v, [si + 2])
                      - plsc.load_gather(group_v, [di + 2]))
                d2 = dx * dx + dy * dy + dz * dz
                bits = plsc.bitcast(d2, jnp.int32)
                bits = 0x5F3759DF - lax.shift_right_logical(bits, 1)
                y = plsc.bitcast(bits, jnp.float32)
                y = y * (1.5 - 0.5 * d2 * y * y)
                y = y * (1.5 - 0.5 * d2 * y * y)
                y = y * (1.5 - 0.5 * d2 * y * y)
                ew_v[pl.ds(off, _LANES)] = d2 * y
                rows = off * 3 + lane_iota3
                plsc.store_scatter(ev_v, [rows], dx * y)
                plsc.store_scatter(ev_v, [rows + 1], dy * y)
                plsc.store_scatter(ev_v, [rows + 2], dz * y)
                return carry2

            lax.fori_loop(0, _CHUNK // _LANES, vec_body, 0)
            pltpu.sync_copy(ew_v, ew_hbm.at[pl.ds(base, _CHUNK)])
            pltpu.sync_copy(ev_v, ev_hbm.at[pl.ds(base * 3, _CHUNK * 3)])
            return carry

        lax.fori_loop(0, n_chunks, chunk_body, 0)

    k = pl.kernel(
        body,
        out_type=(
            jax.ShapeDtypeStruct((n_edges,), jnp.float32),
            jax.ShapeDtypeStruct((n_edges * 3,), jnp.float32),
        ),
        mesh=mesh,
        compiler_params=pltpu.CompilerParams(needs_layout_passes=False),
        scratch_types=[
            pltpu.VMEM(node_flat.shape, jnp.float32),
            pltpu.VMEM(group_flat.shape, jnp.float32),
            pltpu.VMEM((_CHUNK,), jnp.int32),
            pltpu.VMEM((_CHUNK,), jnp.int32),
            pltpu.VMEM((_CHUNK,), jnp.float32),
            pltpu.VMEM((_CHUNK * 3,), jnp.float32),
        ],
    )
    return k(src, dst, node_flat, group_flat)


_TC_BLK = 512


def _tc_rbf_body(ew_ref, means_ref, betas_ref, w_ref, b_ref, attr_ref):
    alpha = 5.0 / (CUTOFF_UPPER - CUTOFF_LOWER)
    d = ew_ref[...]
    cutoff = 0.5 * (jnp.cos(d * (math.pi / CUTOFF_UPPER)) + 1.0)
    cutoff = jnp.where(d < CUTOFF_UPPER, cutoff, 0.0)
    t = jnp.exp(alpha * (CUTOFF_LOWER - d)) - means_ref[...]
    rbf = cutoff * jnp.exp(-betas_ref[...] * t * t)
    attr_ref[...] = (
        jnp.dot(rbf, w_ref[...], preferred_element_type=jnp.float32)
        + b_ref[...]
    )


def _tc_rbf_linear(ew, means, betas, W, b):
    n_edges = ew.shape[0]
    num_rbf, hidden = W.shape
    grid = (n_edges // _TC_BLK,)
    return pl.pallas_call(
        _tc_rbf_body,
        grid=grid,
        in_specs=[
            pl.BlockSpec((_TC_BLK, 1), lambda i: (i, 0)),
            pl.BlockSpec((1, num_rbf), lambda i: (0, 0)),
            pl.BlockSpec((1, num_rbf), lambda i: (0, 0)),
            pl.BlockSpec((num_rbf, hidden), lambda i: (0, 0)),
            pl.BlockSpec((1, hidden), lambda i: (0, 0)),
        ],
        out_specs=pl.BlockSpec((_TC_BLK, hidden), lambda i: (i, 0)),
        out_shape=jax.ShapeDtypeStruct((n_edges, hidden), jnp.float32),
    )(
        ew.reshape(n_edges, 1),
        means.reshape(1, num_rbf),
        betas.reshape(1, num_rbf),
        W,
        b.reshape(1, hidden),
    )


def kernel(edge_index, node_pos, group_pos, means, betas, W, b):
    src = edge_index[0]
    dst = edge_index[1]
    edge_weight, ev_flat = _sc_gather_kernel(
        src, dst, node_pos.reshape(-1), group_pos.reshape(-1))
    edge_vec = ev_flat.reshape(-1, 3)
    edge_attr = _tc_rbf_linear(edge_weight, means, betas, W, b)
    return (edge_index, edge_weight, edge_attr, edge_vec)


# speedup vs baseline: 1.7413x; 1.0812x over previous
import functools
import math

import jax
import jax.numpy as jnp
from jax import lax
from jax.experimental import pallas as pl
from jax.experimental.pallas import tpu as pltpu
from jax.experimental.pallas import tpu_sc as plsc

CUTOFF_LOWER = 0.0
CUTOFF_UPPER = 10.0

_NUM_CORES = 2
_NUM_SUBCORES = 16
_NW = _NUM_CORES * _NUM_SUBCORES
_LANES = 16
_CHUNK = 2000


def _sc_gather_kernel(src, dst, node_flat, group_flat):
    n_edges = src.shape[0]
    edges_per_w = n_edges // _NW
    n_chunks = edges_per_w // _CHUNK

    mesh = plsc.VectorSubcoreMesh(core_axis_name="c", subcore_axis_name="s")

    def body(src_hbm, dst_hbm, node_hbm, group_hbm, ew_hbm, ev_hbm,
             node_v, group_v, src_v, dst_v, ew_v, ev_v):
        cid = lax.axis_index("c")
        sid = lax.axis_index("s")
        wid = sid * _NUM_CORES + cid
        base_w = wid * edges_per_w
        pltpu.sync_copy(node_hbm, node_v)
        pltpu.sync_copy(group_hbm, group_v)

        lane_iota3 = lax.iota(jnp.int32, _LANES) * 3

        def chunk_body(k, carry):
            base = base_w + k * _CHUNK
            pltpu.sync_copy(src_hbm.at[pl.ds(base, _CHUNK)], src_v)
            pltpu.sync_copy(dst_hbm.at[pl.ds(base, _CHUNK)], dst_v)

            def vec_body(j, carry2):
                off = j * _LANES
                si = src_v[pl.ds(off, _LANES)] * 3
                di = dst_v[pl.ds(off, _LANES)] * 3
                dx = (plsc.load_gather(node_v, [si])
                      - plsc.load_gather(group_v, [di]))
                dy = (plsc.load_gather(node_v, [si + 1])
                      - plsc.load_gather(group_v, [di + 1]))
                dz = (plsc.load_gather(node_v, [si + 2])
                      - plsc.load_gather(group_v, [di + 2]))
                d2 = dx * dx + dy * dy + dz * dz
                bits = plsc.bitcast(d2, jnp.int32)
                bits = 0x5F3759DF - lax.shift_right_logical(bits, 1)
                y = plsc.bitcast(bits, jnp.float32)
                y = y * (1.5 - 0.5 * d2 * y * y)
                y = y * (1.5 - 0.5 * d2 * y * y)
                y = y * (1.5 - 0.5 * d2 * y * y)
                ew_v[pl.ds(off, _LANES)] = d2 * y
                rows = off * 3 + lane_iota3
                plsc.store_scatter(ev_v, [rows], dx * y)
                plsc.store_scatter(ev_v, [rows + 1], dy * y)
                plsc.store_scatter(ev_v, [rows + 2], dz * y)
                return carry2

            lax.fori_loop(0, _CHUNK // _LANES, vec_body, 0)
            pltpu.sync_copy(ew_v, ew_hbm.at[pl.ds(base, _CHUNK)])
            pltpu.sync_copy(ev_v, ev_hbm.at[pl.ds(base * 3, _CHUNK * 3)])
            return carry

        lax.fori_loop(0, n_chunks, chunk_body, 0)

    k = pl.kernel(
        body,
        out_type=(
            jax.ShapeDtypeStruct((n_edges,), jnp.float32),
            jax.ShapeDtypeStruct((n_edges * 3,), jnp.float32),
        ),
        mesh=mesh,
        compiler_params=pltpu.CompilerParams(needs_layout_passes=False),
        scratch_types=[
            pltpu.VMEM(node_flat.shape, jnp.float32),
            pltpu.VMEM(group_flat.shape, jnp.float32),
            pltpu.VMEM((_CHUNK,), jnp.int32),
            pltpu.VMEM((_CHUNK,), jnp.int32),
            pltpu.VMEM((_CHUNK,), jnp.float32),
            pltpu.VMEM((_CHUNK * 3,), jnp.float32),
        ],
    )
    return k(src, dst, node_flat, group_flat)


_TC_BLK = 4000


def _tc_rbf_body(ew_ref, means_ref, betas_ref, w_ref, b_ref, attr_ref):
    alpha = 5.0 / (CUTOFF_UPPER - CUTOFF_LOWER)
    d = ew_ref[...]
    cutoff = 0.5 * (jnp.cos(d * (math.pi / CUTOFF_UPPER)) + 1.0)
    cutoff = jnp.where(d < CUTOFF_UPPER, cutoff, 0.0)
    t = jnp.exp(alpha * (CUTOFF_LOWER - d)) - means_ref[...]
    rbf = cutoff * jnp.exp(-betas_ref[...] * t * t)
    attr_ref[...] = (
        jnp.dot(rbf, w_ref[...], preferred_element_type=jnp.float32)
        + b_ref[...]
    )


def _tc_rbf_linear(ew, means, betas, W, b):
    n_edges = ew.shape[0]
    num_rbf, hidden = W.shape
    grid = (n_edges // _TC_BLK,)
    return pl.pallas_call(
        _tc_rbf_body,
        grid=grid,
        in_specs=[
            pl.BlockSpec((_TC_BLK, 1), lambda i: (i, 0)),
            pl.BlockSpec((1, num_rbf), lambda i: (0, 0)),
            pl.BlockSpec((1, num_rbf), lambda i: (0, 0)),
            pl.BlockSpec((num_rbf, hidden), lambda i: (0, 0)),
            pl.BlockSpec((1, hidden), lambda i: (0, 0)),
        ],
        out_specs=pl.BlockSpec((_TC_BLK, hidden), lambda i: (i, 0)),
        out_shape=jax.ShapeDtypeStruct((n_edges, hidden), jnp.float32),
    )(
        ew.reshape(n_edges, 1),
        means.reshape(1, num_rbf),
        betas.reshape(1, num_rbf),
        W,
        b.reshape(1, hidden),
    )


def kernel(edge_index, node_pos, group_pos, means, betas, W, b):
    src = edge_index[0]
    dst = edge_index[1]
    edge_weight, ev_flat = _sc_gather_kernel(
        src, dst, node_pos.reshape(-1), group_pos.reshape(-1))
    edge_vec = ev_flat.reshape(-1, 3)
    edge_attr = _tc_rbf_linear(edge_weight, means, betas, W, b)
    return (edge_index, edge_weight, edge_attr, edge_vec)


# speedup vs baseline: 2.3981x; 1.3772x over previous
import functools
import math

import jax
import jax.numpy as jnp
from jax import lax
from jax.experimental import pallas as pl
from jax.experimental.pallas import tpu as pltpu
from jax.experimental.pallas import tpu_sc as plsc

CUTOFF_LOWER = 0.0
CUTOFF_UPPER = 10.0

_NUM_CORES = 2
_NUM_SUBCORES = 16
_NW = _NUM_CORES * _NUM_SUBCORES
_LANES = 16
_CHUNK = 2000


_COS_COEFFS = (1.0, -0.49999985098838806, 0.041666463017463684,
               -0.0013887732056900859, 2.4769053197815083e-05,
               -2.707544979330123e-07, 1.7243751981865785e-09)


def _sc_gather_kernel(src, dst, node_flat, group_flat):
    n_edges = src.shape[0]
    edges_per_w = n_edges // _NW
    n_chunks = edges_per_w // _CHUNK
    alpha = 5.0 / (CUTOFF_UPPER - CUTOFF_LOWER)
    u_scale = (math.pi / CUTOFF_UPPER) ** 2
    cut2 = CUTOFF_UPPER * CUTOFF_UPPER

    mesh = plsc.VectorSubcoreMesh(core_axis_name="c", subcore_axis_name="s")

    def body(src_hbm, dst_hbm, node_hbm, group_hbm,
             ew_hbm, cut_hbm, ex_hbm, ev_hbm,
             node_v, group_v, src_v, dst_v, ew_v, cut_v, ex_v, ev_v):
        cid = lax.axis_index("c")
        sid = lax.axis_index("s")
        wid = sid * _NUM_CORES + cid
        base_w = wid * edges_per_w
        pltpu.sync_copy(node_hbm, node_v)
        pltpu.sync_copy(group_hbm, group_v)

        lane_iota3 = lax.iota(jnp.int32, _LANES) * 3

        def chunk_body(k, carry):
            base = base_w + k * _CHUNK
            pltpu.sync_copy(src_hbm.at[pl.ds(base, _CHUNK)], src_v)
            pltpu.sync_copy(dst_hbm.at[pl.ds(base, _CHUNK)], dst_v)

            def vec_body(j, carry2):
                off = j * _LANES
                si = src_v[pl.ds(off, _LANES)] * 3
                di = dst_v[pl.ds(off, _LANES)] * 3
                dx = (plsc.load_gather(node_v, [si])
                      - plsc.load_gather(group_v, [di]))
                dy = (plsc.load_gather(node_v, [si + 1])
                      - plsc.load_gather(group_v, [di + 1]))
                dz = (plsc.load_gather(node_v, [si + 2])
                      - plsc.load_gather(group_v, [di + 2]))
                d2 = dx * dx + dy * dy + dz * dz
                bits = plsc.bitcast(d2, jnp.int32)
                bits = 0x5F3759DF - lax.shift_right_logical(bits, 1)
                y = plsc.bitcast(bits, jnp.float32)
                y = y * (1.5 - 0.5 * d2 * y * y)
                y = y * (1.5 - 0.5 * d2 * y * y)
                y = y * (1.5 - 0.5 * d2 * y * y)
                d = d2 * y
                ew_v[pl.ds(off, _LANES)] = d
                u = u_scale * jnp.minimum(d2, cut2)
                cosv = jnp.full((_LANES,), _COS_COEFFS[6], jnp.float32)
                cosv = cosv * u + _COS_COEFFS[5]
                cosv = cosv * u + _COS_COEFFS[4]
                cosv = cosv * u + _COS_COEFFS[3]
                cosv = cosv * u + _COS_COEFFS[2]
                cosv = cosv * u + _COS_COEFFS[1]
                cosv = cosv * u + _COS_COEFFS[0]
                cut = jnp.where(d2 < cut2, 0.5 * (cosv + 1.0), 0.0)
                cut_v[pl.ds(off, _LANES)] = cut
                ex_v[pl.ds(off, _LANES)] = jnp.exp(-alpha * d)
                rows = off * 3 + lane_iota3
                plsc.store_scatter(ev_v, [rows], dx * y)
                plsc.store_scatter(ev_v, [rows + 1], dy * y)
                plsc.store_scatter(ev_v, [rows + 2], dz * y)
                return carry2

            lax.fori_loop(0, _CHUNK // _LANES, vec_body, 0)
            pltpu.sync_copy(ew_v, ew_hbm.at[pl.ds(base, _CHUNK)])
            pltpu.sync_copy(cut_v, cut_hbm.at[pl.ds(base, _CHUNK)])
            pltpu.sync_copy(ex_v, ex_hbm.at[pl.ds(base, _CHUNK)])
            pltpu.sync_copy(ev_v, ev_hbm.at[pl.ds(base * 3, _CHUNK * 3)])
            return carry

        lax.fori_loop(0, n_chunks, chunk_body, 0)

    k = pl.kernel(
        body,
        out_type=(
            jax.ShapeDtypeStruct((n_edges,), jnp.float32),
            jax.ShapeDtypeStruct((n_edges,), jnp.float32),
            jax.ShapeDtypeStruct((n_edges,), jnp.float32),
            jax.ShapeDtypeStruct((n_edges * 3,), jnp.float32),
        ),
        mesh=mesh,
        compiler_params=pltpu.CompilerParams(needs_layout_passes=False),
        scratch_types=[
            pltpu.VMEM(node_flat.shape, jnp.float32),
            pltpu.VMEM(group_flat.shape, jnp.float32),
            pltpu.VMEM((_CHUNK,), jnp.int32),
            pltpu.VMEM((_CHUNK,), jnp.int32),
            pltpu.VMEM((_CHUNK,), jnp.float32),
            pltpu.VMEM((_CHUNK,), jnp.float32),
            pltpu.VMEM((_CHUNK,), jnp.float32),
            pltpu.VMEM((_CHUNK * 3,), jnp.float32),
        ],
    )
    return k(src, dst, node_flat, group_flat)


_TC_BLK = 4000


def _tc_rbf_body(cut_ref, ex_ref, means_ref, betas_ref, w_ref, b_ref,
                 attr_ref):
    t = ex_ref[...] - means_ref[...]
    rbf = cut_ref[...] * jnp.exp(-betas_ref[...] * t * t)
    attr_ref[...] = (
        jnp.dot(rbf, w_ref[...], preferred_element_type=jnp.float32)
        + b_ref[...]
    )


def _tc_rbf_linear(cut, ex, means, betas, W, b):
    n_edges = cut.shape[0]
    num_rbf, hidden = W.shape
    grid = (n_edges // _TC_BLK,)
    return pl.pallas_call(
        _tc_rbf_body,
        grid=grid,
        in_specs=[
            pl.BlockSpec((_TC_BLK, 1), lambda i: (i, 0)),
            pl.BlockSpec((_TC_BLK, 1), lambda i: (i, 0)),
            pl.BlockSpec((1, num_rbf), lambda i: (0, 0)),
            pl.BlockSpec((1, num_rbf), lambda i: (0, 0)),
            pl.BlockSpec((num_rbf, hidden), lambda i: (0, 0)),
            pl.BlockSpec((1, hidden), lambda i: (0, 0)),
        ],
        out_specs=pl.BlockSpec((_TC_BLK, hidden), lambda i: (i, 0)),
        out_shape=jax.ShapeDtypeStruct((n_edges, hidden), jnp.float32),
    )(
        cut.reshape(n_edges, 1),
        ex.reshape(n_edges, 1),
        means.reshape(1, num_rbf),
        betas.reshape(1, num_rbf),
        W,
        b.reshape(1, hidden),
    )


def kernel(edge_index, node_pos, group_pos, means, betas, W, b):
    src = edge_index[0]
    dst = edge_index[1]
    edge_weight, cut, ex, ev_flat = _sc_gather_kernel(
        src, dst, node_pos.reshape(-1), group_pos.reshape(-1))
    edge_vec = ev_flat.reshape(-1, 3)
    edge_attr = _tc_rbf_linear(cut, ex, means, betas, W, b)
    return (edge_index, edge_weight, edge_attr, edge_vec)


# speedup vs baseline: 3.8732x; 1.6151x over previous
import functools
import math

import jax
import jax.numpy as jnp
from jax import lax
from jax.experimental import pallas as pl
from jax.experimental.pallas import tpu as pltpu
from jax.experimental.pallas import tpu_sc as plsc

CUTOFF_LOWER = 0.0
CUTOFF_UPPER = 10.0

_NUM_CORES = 2
_NUM_SUBCORES = 16
_NW = _NUM_CORES * _NUM_SUBCORES
_LANES = 16
_CHUNK = 2000


_COS_COEFFS = (1.0, -0.49999985098838806, 0.041666463017463684,
               -0.0013887732056900859, 2.4769053197815083e-05,
               -2.707544979330123e-07, 1.7243751981865785e-09)


def _sc_gather_kernel(src, dst, node_flat, group_flat):
    n_edges = src.shape[0]
    edges_per_w = n_edges // _NW
    n_chunks = edges_per_w // _CHUNK
    alpha = 5.0 / (CUTOFF_UPPER - CUTOFF_LOWER)
    u_scale = (math.pi / CUTOFF_UPPER) ** 2
    cut2 = CUTOFF_UPPER * CUTOFF_UPPER

    mesh = plsc.VectorSubcoreMesh(core_axis_name="c", subcore_axis_name="s")

    def body(src_hbm, dst_hbm, node_hbm, group_hbm,
             ew_hbm, cut_hbm, ex_hbm, ev_hbm,
             node_v, group_v, src_v, dst_v, ew_v, cut_v, ex_v, ev_v):
        cid = lax.axis_index("c")
        sid = lax.axis_index("s")
        wid = sid * _NUM_CORES + cid
        base_w = wid * edges_per_w
        pltpu.sync_copy(node_hbm, node_v)
        pltpu.sync_copy(group_hbm, group_v)

        lane_iota3 = lax.iota(jnp.int32, _LANES) * 3

        def chunk_body(k, carry):
            base = base_w + k * _CHUNK
            pltpu.sync_copy(src_hbm.at[pl.ds(base, _CHUNK)], src_v)
            pltpu.sync_copy(dst_hbm.at[pl.ds(base, _CHUNK)], dst_v)

            def vec_body(j, carry2):
                off = j * _LANES
                si = src_v[pl.ds(off, _LANES)] * 3
                di = dst_v[pl.ds(off, _LANES)] * 3
                dx = (plsc.load_gather(node_v, [si])
                      - plsc.load_gather(group_v, [di]))
                dy = (plsc.load_gather(node_v, [si + 1])
                      - plsc.load_gather(group_v, [di + 1]))
                dz = (plsc.load_gather(node_v, [si + 2])
                      - plsc.load_gather(group_v, [di + 2]))
                d2 = dx * dx + dy * dy + dz * dz
                bits = plsc.bitcast(d2, jnp.int32)
                bits = 0x5F3759DF - lax.shift_right_logical(bits, 1)
                y = plsc.bitcast(bits, jnp.float32)
                y = y * (1.5 - 0.5 * d2 * y * y)
                y = y * (1.5 - 0.5 * d2 * y * y)
                y = y * (1.5 - 0.5 * d2 * y * y)
                d = d2 * y
                ew_v[pl.ds(off, _LANES)] = d
                u = u_scale * jnp.minimum(d2, cut2)
                cosv = jnp.full((_LANES,), _COS_COEFFS[6], jnp.float32)
                cosv = cosv * u + _COS_COEFFS[5]
                cosv = cosv * u + _COS_COEFFS[4]
                cosv = cosv * u + _COS_COEFFS[3]
                cosv = cosv * u + _COS_COEFFS[2]
                cosv = cosv * u + _COS_COEFFS[1]
                cosv = cosv * u + _COS_COEFFS[0]
                cut = jnp.where(d2 < cut2, 0.5 * (cosv + 1.0), 0.0)
                cut_v[pl.ds(off, _LANES)] = cut
                ex_v[pl.ds(off, _LANES)] = jnp.exp(-alpha * d)
                rows = off * 3 + lane_iota3
                plsc.store_scatter(ev_v, [rows], dx * y)
                plsc.store_scatter(ev_v, [rows + 1], dy * y)
                plsc.store_scatter(ev_v, [rows + 2], dz * y)
                return carry2

            lax.fori_loop(0, _CHUNK // _LANES, vec_body, 0)
            pltpu.sync_copy(ew_v, ew_hbm.at[pl.ds(base, _CHUNK)])
            pltpu.sync_copy(cut_v, cut_hbm.at[pl.ds(base, _CHUNK)])
            pltpu.sync_copy(ex_v, ex_hbm.at[pl.ds(base, _CHUNK)])
            pltpu.sync_copy(ev_v, ev_hbm.at[pl.ds(base * 3, _CHUNK * 3)])
            return carry

        lax.fori_loop(0, n_chunks, chunk_body, 0)

    k = pl.kernel(
        body,
        out_type=(
            jax.ShapeDtypeStruct((n_edges,), jnp.float32),
            jax.ShapeDtypeStruct((n_edges,), jnp.float32),
            jax.ShapeDtypeStruct((n_edges,), jnp.float32),
            jax.ShapeDtypeStruct((n_edges * 3,), jnp.float32),
        ),
        mesh=mesh,
        compiler_params=pltpu.CompilerParams(needs_layout_passes=False),
        scratch_types=[
            pltpu.VMEM(node_flat.shape, jnp.float32),
            pltpu.VMEM(group_flat.shape, jnp.float32),
            pltpu.VMEM((_CHUNK,), jnp.int32),
            pltpu.VMEM((_CHUNK,), jnp.int32),
            pltpu.VMEM((_CHUNK,), jnp.float32),
            pltpu.VMEM((_CHUNK,), jnp.float32),
            pltpu.VMEM((_CHUNK,), jnp.float32),
            pltpu.VMEM((_CHUNK * 3,), jnp.float32),
        ],
    )
    return k(src, dst, node_flat, group_flat)


_TC_BLK = 4000


def _tc_rbf_body(cut_ref, ex_ref, means_ref, betas_ref, w_ref, b_ref,
                 attr_ref):
    ex_row = ex_ref[...].reshape(1, ex_ref.shape[-1])
    cut_row = cut_ref[...].reshape(1, cut_ref.shape[-1])
    t = ex_row - means_ref[...]
    rbf = cut_row * jnp.exp(-betas_ref[...] * t * t)
    attr_ref[...] = lax.dot_general(
        rbf, w_ref[...],
        dimension_numbers=(((0,), (0,)), ((), ())),
        preferred_element_type=jnp.float32,
    ) + b_ref[...]


def _tc_rbf_linear(cut, ex, means, betas, W, b):
    n_edges = cut.shape[0]
    num_rbf, hidden = W.shape
    grid = (n_edges // _TC_BLK,)
    return pl.pallas_call(
        _tc_rbf_body,
        grid=grid,
        in_specs=[
            pl.BlockSpec((1, 1, _TC_BLK), lambda i: (i, 0, 0)),
            pl.BlockSpec((1, 1, _TC_BLK), lambda i: (i, 0, 0)),
            pl.BlockSpec((num_rbf, 1), lambda i: (0, 0)),
            pl.BlockSpec((num_rbf, 1), lambda i: (0, 0)),
            pl.BlockSpec((num_rbf, hidden), lambda i: (0, 0)),
            pl.BlockSpec((1, hidden), lambda i: (0, 0)),
        ],
        out_specs=pl.BlockSpec((_TC_BLK, hidden), lambda i: (i, 0)),
        out_shape=jax.ShapeDtypeStruct((n_edges, hidden), jnp.float32),
    )(
        cut.reshape(-1, 1, _TC_BLK),
        ex.reshape(-1, 1, _TC_BLK),
        means.reshape(num_rbf, 1),
        betas.reshape(num_rbf, 1),
        W,
        b.reshape(1, hidden),
    )


def kernel(edge_index, node_pos, group_pos, means, betas, W, b):
    src = edge_index[0]
    dst = edge_index[1]
    edge_weight, cut, ex, ev_flat = _sc_gather_kernel(
        src, dst, node_pos.reshape(-1), group_pos.reshape(-1))
    edge_vec = ev_flat.reshape(-1, 3)
    edge_attr = _tc_rbf_linear(cut, ex, means, betas, W, b)
    return (edge_index, edge_weight, edge_attr, edge_vec)


# speedup vs baseline: 4.2634x; 1.1007x over previous
import functools
import math

import jax
import jax.numpy as jnp
from jax import lax
from jax.experimental import pallas as pl
from jax.experimental.pallas import tpu as pltpu
from jax.experimental.pallas import tpu_sc as plsc

CUTOFF_LOWER = 0.0
CUTOFF_UPPER = 10.0

_NUM_CORES = 2
_NUM_SUBCORES = 16
_NW = _NUM_CORES * _NUM_SUBCORES
_LANES = 16
_CHUNK = 2000


_COS_COEFFS = (1.0, -0.49999985098838806, 0.041666463017463684,
               -0.0013887732056900859, 2.4769053197815083e-05,
               -2.707544979330123e-07, 1.7243751981865785e-09)


def _sc_gather_kernel(edge_index, node_flat, group_flat):
    n_edges = edge_index.shape[0] // 2
    edges_per_w = n_edges // _NW
    n_chunks = edges_per_w // _CHUNK
    n_rows = n_edges // _CHUNK
    alpha = 5.0 / (CUTOFF_UPPER - CUTOFF_LOWER)
    u_scale = (math.pi / CUTOFF_UPPER) ** 2
    cut2 = CUTOFF_UPPER * CUTOFF_UPPER

    mesh = plsc.VectorSubcoreMesh(core_axis_name="c", subcore_axis_name="s")

    def body(ei_hbm, node_hbm, group_hbm,
             ew_hbm, cut_hbm, ex_hbm, ev_hbm,
             node_v, group_v, src_v, dst_v, ew_v, cut_v, ex_v, ev_v):
        cid = lax.axis_index("c")
        sid = lax.axis_index("s")
        wid = sid * _NUM_CORES + cid
        base_w = wid * edges_per_w
        pltpu.sync_copy(node_hbm, node_v)
        pltpu.sync_copy(group_hbm, group_v)

        lane_iota3 = lax.iota(jnp.int32, _LANES) * 3

        def chunk_body(k, carry):
            base = base_w + k * _CHUNK
            row = wid * n_chunks + k
            pltpu.sync_copy(ei_hbm.at[pl.ds(base, _CHUNK)], src_v)
            pltpu.sync_copy(ei_hbm.at[pl.ds(n_edges + base, _CHUNK)], dst_v)

            def vec_body(j, carry2):
                off = j * _LANES
                si = src_v[pl.ds(off, _LANES)] * 3
                di = dst_v[pl.ds(off, _LANES)] * 3
                dx = (plsc.load_gather(node_v, [si])
                      - plsc.load_gather(group_v, [di]))
                dy = (plsc.load_gather(node_v, [si + 1])
                      - plsc.load_gather(group_v, [di + 1]))
                dz = (plsc.load_gather(node_v, [si + 2])
                      - plsc.load_gather(group_v, [di + 2]))
                d2 = dx * dx + dy * dy + dz * dz
                bits = plsc.bitcast(d2, jnp.int32)
                bits = 0x5F3759DF - lax.shift_right_logical(bits, 1)
                y = plsc.bitcast(bits, jnp.float32)
                y = y * (1.5 - 0.5 * d2 * y * y)
                y = y * (1.5 - 0.5 * d2 * y * y)
                y = y * (1.5 - 0.5 * d2 * y * y)
                d = d2 * y
                ew_v[pl.ds(off, _LANES)] = d
                u = u_scale * jnp.minimum(d2, cut2)
                cosv = jnp.full((_LANES,), _COS_COEFFS[6], jnp.float32)
                cosv = cosv * u + _COS_COEFFS[5]
                cosv = cosv * u + _COS_COEFFS[4]
                cosv = cosv * u + _COS_COEFFS[3]
                cosv = cosv * u + _COS_COEFFS[2]
                cosv = cosv * u + _COS_COEFFS[1]
                cosv = cosv * u + _COS_COEFFS[0]
                cut = jnp.where(d2 < cut2, 0.5 * (cosv + 1.0), 0.0)
                cut_v[pl.ds(off, _LANES)] = cut
                ex_v[pl.ds(off, _LANES)] = jnp.exp(-alpha * d)
                rows = off * 3 + lane_iota3
                plsc.store_scatter(ev_v, [rows], dx * y)
                plsc.store_scatter(ev_v, [rows + 1], dy * y)
                plsc.store_scatter(ev_v, [rows + 2], dz * y)
                return carry2

            lax.fori_loop(0, _CHUNK // _LANES, vec_body, 0)
            pltpu.sync_copy(ew_v, ew_hbm.at[pl.ds(base, _CHUNK)])
            pltpu.sync_copy(cut_v, cut_hbm.at[row, 0])
            pltpu.sync_copy(ex_v, ex_hbm.at[row, 0])
            pltpu.sync_copy(ev_v, ev_hbm.at[pl.ds(base * 3, _CHUNK * 3)])
            return carry

        lax.fori_loop(0, n_chunks, chunk_body, 0)

    k = pl.kernel(
        body,
        out_type=(
            jax.ShapeDtypeStruct((n_edges,), jnp.float32),
            jax.ShapeDtypeStruct((n_rows, 1, _CHUNK), jnp.float32),
            jax.ShapeDtypeStruct((n_rows, 1, _CHUNK), jnp.float32),
            jax.ShapeDtypeStruct((n_edges * 3,), jnp.float32),
        ),
        mesh=mesh,
        compiler_params=pltpu.CompilerParams(needs_layout_passes=False),
        scratch_types=[
            pltpu.VMEM(node_flat.shape, jnp.float32),
            pltpu.VMEM(group_flat.shape, jnp.float32),
            pltpu.VMEM((_CHUNK,), jnp.int32),
            pltpu.VMEM((_CHUNK,), jnp.int32),
            pltpu.VMEM((_CHUNK,), jnp.float32),
            pltpu.VMEM((_CHUNK,), jnp.float32),
            pltpu.VMEM((_CHUNK,), jnp.float32),
            pltpu.VMEM((_CHUNK * 3,), jnp.float32),
        ],
    )
    return k(edge_index, node_flat, group_flat)


_TC_BLK = _CHUNK


def _tc_rbf_body(cut_ref, ex_ref, means_ref, betas_ref, w_ref, b_ref,
                 attr_ref):
    ex_row = ex_ref[...].reshape(1, ex_ref.shape[-1])
    cut_row = cut_ref[...].reshape(1, cut_ref.shape[-1])
    t = ex_row - means_ref[...]
    rbf = cut_row * jnp.exp(-betas_ref[...] * t * t)
    attr_ref[...] = lax.dot_general(
        rbf, w_ref[...],
        dimension_numbers=(((0,), (0,)), ((), ())),
        preferred_element_type=jnp.float32,
    ) + b_ref[...]


def _tc_rbf_linear(cut, ex, means, betas, W, b):
    n_edges = cut.shape[0] * cut.shape[2]
    num_rbf, hidden = W.shape
    grid = (n_edges // _TC_BLK,)
    return pl.pallas_call(
        _tc_rbf_body,
        grid=grid,
        in_specs=[
            pl.BlockSpec((1, 1, _TC_BLK), lambda i: (i, 0, 0)),
            pl.BlockSpec((1, 1, _TC_BLK), lambda i: (i, 0, 0)),
            pl.BlockSpec((num_rbf, 1), lambda i: (0, 0)),
            pl.BlockSpec((num_rbf, 1), lambda i: (0, 0)),
            pl.BlockSpec((num_rbf, hidden), lambda i: (0, 0)),
            pl.BlockSpec((1, hidden), lambda i: (0, 0)),
        ],
        out_specs=pl.BlockSpec((_TC_BLK, hidden), lambda i: (i, 0)),
        out_shape=jax.ShapeDtypeStruct((n_edges, hidden), jnp.float32),
    )(
        cut,
        ex,
        means.reshape(num_rbf, 1),
        betas.reshape(num_rbf, 1),
        W,
        b.reshape(1, hidden),
    )


def kernel(edge_index, node_pos, group_pos, means, betas, W, b):
    edge_weight, cut, ex, ev_flat = _sc_gather_kernel(
        edge_index.reshape(-1), node_pos.reshape(-1), group_pos.reshape(-1))
    edge_vec = ev_flat.reshape(-1, 3)
    edge_attr = _tc_rbf_linear(cut, ex, means, betas, W, b)
    return (edge_index, edge_weight, edge_attr, edge_vec)


# speedup vs baseline: 4.3469x; 1.0196x over previous
import functools
import math

import jax
import jax.numpy as jnp
from jax import lax
from jax.experimental import pallas as pl
from jax.experimental.pallas import tpu as pltpu
from jax.experimental.pallas import tpu_sc as plsc

CUTOFF_LOWER = 0.0
CUTOFF_UPPER = 10.0

_NUM_CORES = 2
_NUM_SUBCORES = 16
_NW = _NUM_CORES * _NUM_SUBCORES
_LANES = 16
_CHUNK = 2000


_COS_COEFFS = (1.0, -0.49999985098838806, 0.041666463017463684,
               -0.0013887732056900859, 2.4769053197815083e-05,
               -2.707544979330123e-07, 1.7243751981865785e-09)


def _sc_gather_kernel(edge_index, node_flat, group_flat):
    n_edges = edge_index.shape[0] // 2
    edges_per_w = n_edges // _NW
    n_chunks = edges_per_w // _CHUNK
    n_rows = n_edges // _CHUNK
    alpha = 5.0 / (CUTOFF_UPPER - CUTOFF_LOWER)
    u_scale = (math.pi / CUTOFF_UPPER) ** 2
    cut2 = CUTOFF_UPPER * CUTOFF_UPPER

    mesh = plsc.VectorSubcoreMesh(core_axis_name="c", subcore_axis_name="s")

    def body(ei_hbm, node_hbm, group_hbm,
             ew_hbm, cut_hbm, ex_hbm, ev_hbm,
             node_v, group_v,
             src_b0, dst_b0, ew_b0, cut_b0, ex_b0, ev_b0,
             src_b1, dst_b1, ew_b1, cut_b1, ex_b1, ev_b1,
             in_sem0, in_sem1, out_sem0, out_sem1):
        cid = lax.axis_index("c")
        sid = lax.axis_index("s")
        wid = sid * _NUM_CORES + cid
        base_w = wid * edges_per_w
        pltpu.sync_copy(node_hbm, node_v)
        pltpu.sync_copy(group_hbm, group_v)

        srcs = (src_b0, src_b1)
        dsts = (dst_b0, dst_b1)
        ews = (ew_b0, ew_b1)
        cuts = (cut_b0, cut_b1)
        exs = (ex_b0, ex_b1)
        evs = (ev_b0, ev_b1)
        in_sems = (in_sem0, in_sem1)
        out_sems = (out_sem0, out_sem1)

        lane_iota = lax.iota(jnp.int32, _LANES)

        def start_in(k, b):
            base = base_w + k * _CHUNK
            return (
                pltpu.async_copy(ei_hbm.at[pl.ds(base, _CHUNK)],
                                 srcs[b], in_sems[b]),
                pltpu.async_copy(ei_hbm.at[pl.ds(n_edges + base, _CHUNK)],
                                 dsts[b], in_sems[b]),
            )

        def start_out(k, b):
            base = base_w + k * _CHUNK
            row = wid * n_chunks + k
            return (
                pltpu.async_copy(ews[b], ew_hbm.at[pl.ds(base, _CHUNK)],
                                 out_sems[b]),
                pltpu.async_copy(cuts[b], cut_hbm.at[row, 0], out_sems[b]),
                pltpu.async_copy(exs[b], ex_hbm.at[row, 0], out_sems[b]),
                pltpu.async_copy(evs[b],
                                 ev_hbm.at[pl.ds(base * 3, _CHUNK * 3)],
                                 out_sems[b]),
            )

        def compute(b):
            src_v, dst_v = srcs[b], dsts[b]
            ew_v, cut_v, ex_v, ev_v = ews[b], cuts[b], exs[b], evs[b]

            def vec_body(j, carry2):
                off = j * _LANES
                si = src_v[pl.ds(off, _LANES)] * 3
                di = dst_v[pl.ds(off, _LANES)] * 3
                dx = (plsc.load_gather(node_v, [si])
                      - plsc.load_gather(group_v, [di]))
                dy = (plsc.load_gather(node_v, [si + 1])
                      - plsc.load_gather(group_v, [di + 1]))
                dz = (plsc.load_gather(node_v, [si + 2])
                      - plsc.load_gather(group_v, [di + 2]))
                d2 = dx * dx + dy * dy + dz * dz
                bits = plsc.bitcast(d2, jnp.int32)
                bits = 0x5F3759DF - lax.shift_right_logical(bits, 1)
                y = plsc.bitcast(bits, jnp.float32)
                y = y * (1.5 - 0.5 * d2 * y * y)
                y = y * (1.5 - 0.5 * d2 * y * y)
                y = y * (1.5 - 0.5 * d2 * y * y)
                d = d2 * y
                ew_v[pl.ds(off, _LANES)] = d
                u = u_scale * jnp.minimum(d2, cut2)
                cosv = jnp.full((_LANES,), _COS_COEFFS[6], jnp.float32)
                cosv = cosv * u + _COS_COEFFS[5]
                cosv = cosv * u + _COS_COEFFS[4]
                cosv = cosv * u + _COS_COEFFS[3]
                cosv = cosv * u + _COS_COEFFS[2]
                cosv = cosv * u + _COS_COEFFS[1]
                cosv = cosv * u + _COS_COEFFS[0]
                cut = jnp.where(d2 < cut2, 0.5 * (cosv + 1.0), 0.0)
                cut_v[pl.ds(off, _LANES)] = cut
                ex_v[pl.ds(off, _LANES)] = jnp.exp(-alpha * d)
                rows = off * 3 + lane_iota * 3
                plsc.store_scatter(ev_v, [rows], dx * y)
                plsc.store_scatter(ev_v, [rows + 1], dy * y)
                plsc.store_scatter(ev_v, [rows + 2], dz * y)
                return carry2

            lax.fori_loop(0, _CHUNK // _LANES, vec_body, 0)

        in_ds = [None] * n_chunks
        out_ds = [None] * n_chunks
        in_ds[0] = start_in(0, 0)
        for k in range(n_chunks):
            b = k & 1
            if k + 1 < n_chunks:
                in_ds[k + 1] = start_in(k + 1, 1 - b)
            for dsc in in_ds[k]:
                dsc.wait()
            if k >= 2:
                for dsc in out_ds[k - 2]:
                    dsc.wait()
            compute(b)
            out_ds[k] = start_out(k, b)
        for k in range(max(0, n_chunks - 2), n_chunks):
            for dsc in out_ds[k]:
                dsc.wait()

    slot = [
        pltpu.VMEM((_CHUNK,), jnp.int32),
        pltpu.VMEM((_CHUNK,), jnp.int32),
        pltpu.VMEM((_CHUNK,), jnp.float32),
        pltpu.VMEM((_CHUNK,), jnp.float32),
        pltpu.VMEM((_CHUNK,), jnp.float32),
        pltpu.VMEM((_CHUNK * 3,), jnp.float32),
    ]
    k = pl.kernel(
        body,
        out_type=(
            jax.ShapeDtypeStruct((n_edges,), jnp.float32),
            jax.ShapeDtypeStruct((n_rows, 1, _CHUNK), jnp.float32),
            jax.ShapeDtypeStruct((n_rows, 1, _CHUNK), jnp.float32),
            jax.ShapeDtypeStruct((n_edges * 3,), jnp.float32),
        ),
        mesh=mesh,
        compiler_params=pltpu.CompilerParams(needs_layout_passes=False),
        scratch_types=[
            pltpu.VMEM(node_flat.shape, jnp.float32),
            pltpu.VMEM(group_flat.shape, jnp.float32),
        ] + slot + [
            pltpu.VMEM((_CHUNK,), jnp.int32),
            pltpu.VMEM((_CHUNK,), jnp.int32),
            pltpu.VMEM((_CHUNK,), jnp.float32),
            pltpu.VMEM((_CHUNK,), jnp.float32),
            pltpu.VMEM((_CHUNK,), jnp.float32),
            pltpu.VMEM((_CHUNK * 3,), jnp.float32),
            pltpu.SemaphoreType.DMA,
            pltpu.SemaphoreType.DMA,
            pltpu.SemaphoreType.DMA,
            pltpu.SemaphoreType.DMA,
        ],
    )
    return k(edge_index, node_flat, group_flat)


_TC_BLK = _CHUNK


def _tc_rbf_body(cut_ref, ex_ref, means_ref, betas_ref, w_ref, b_ref,
                 attr_ref):
    ex_row = ex_ref[...].reshape(1, ex_ref.shape[-1])
    cut_row = cut_ref[...].reshape(1, cut_ref.shape[-1])
    t = ex_row - means_ref[...]
    rbf = cut_row * jnp.exp(-betas_ref[...] * t * t)
    attr_ref[...] = lax.dot_general(
        rbf, w_ref[...],
        dimension_numbers=(((0,), (0,)), ((), ())),
        preferred_element_type=jnp.float32,
    ) + b_ref[...]


def _tc_rbf_linear(cut, ex, means, betas, W, b):
    n_edges = cut.shape[0] * cut.shape[2]
    num_rbf, hidden = W.shape
    grid = (n_edges // _TC_BLK,)
    return pl.pallas_call(
        _tc_rbf_body,
        grid=grid,
        in_specs=[
            pl.BlockSpec((1, 1, _TC_BLK), lambda i: (i, 0, 0)),
            pl.BlockSpec((1, 1, _TC_BLK), lambda i: (i, 0, 0)),
            pl.BlockSpec((num_rbf, 1), lambda i: (0, 0)),
            pl.BlockSpec((num_rbf, 1), lambda i: (0, 0)),
            pl.BlockSpec((num_rbf, hidden), lambda i: (0, 0)),
            pl.BlockSpec((1, hidden), lambda i: (0, 0)),
        ],
        out_specs=pl.BlockSpec((_TC_BLK, hidden), lambda i: (i, 0)),
        out_shape=jax.ShapeDtypeStruct((n_edges, hidden), jnp.float32),
    )(
        cut,
        ex,
        means.reshape(num_rbf, 1),
        betas.reshape(num_rbf, 1),
        W,
        b.reshape(1, hidden),
    )


def kernel(edge_index, node_pos, group_pos, means, betas, W, b):
    edge_weight, cut, ex, ev_flat = _sc_gather_kernel(
        edge_index.reshape(-1), node_pos.reshape(-1), group_pos.reshape(-1))
    edge_vec = ev_flat.reshape(-1, 3)
    edge_attr = _tc_rbf_linear(cut, ex, means, betas, W, b)
    return (edge_index, edge_weight, edge_attr, edge_vec)


# speedup vs baseline: 5.1086x; 1.1752x over previous
import functools
import math

import jax
import jax.numpy as jnp
from jax import lax
from jax.experimental import pallas as pl
from jax.experimental.pallas import tpu as pltpu
from jax.experimental.pallas import tpu_sc as plsc

CUTOFF_LOWER = 0.0
CUTOFF_UPPER = 10.0

_NUM_CORES = 2
_NUM_SUBCORES = 16
_NW = _NUM_CORES * _NUM_SUBCORES
_LANES = 16
_CHUNK = 2000


_COS_COEFFS = (1.0, -0.49999985098838806, 0.041666463017463684,
               -0.0013887732056900859, 2.4769053197815083e-05,
               -2.707544979330123e-07, 1.7243751981865785e-09)


def _sc_gather_kernel(edge_index, node_flat, group_flat):
    n_edges = edge_index.shape[0] // 2
    edges_per_w = n_edges // _NW
    n_chunks = edges_per_w // _CHUNK
    n_rows = n_edges // _CHUNK
    alpha = 5.0 / (CUTOFF_UPPER - CUTOFF_LOWER)
    u_scale = (math.pi / CUTOFF_UPPER) ** 2
    cut2 = CUTOFF_UPPER * CUTOFF_UPPER

    mesh = plsc.VectorSubcoreMesh(core_axis_name="c", subcore_axis_name="s")

    def body(ei_hbm, node_hbm, group_hbm,
             ew_hbm, cut_hbm, ex_hbm, ev_hbm,
             node_v, group_v,
             src_b0, dst_b0, ew_b0, cut_b0, ex_b0, ev_b0,
             src_b1, dst_b1, ew_b1, cut_b1, ex_b1, ev_b1,
             in_sem0, in_sem1, out_sem0, out_sem1):
        cid = lax.axis_index("c")
        sid = lax.axis_index("s")
        wid = sid * _NUM_CORES + cid
        base_w = wid * edges_per_w
        pltpu.sync_copy(node_hbm, node_v)
        pltpu.sync_copy(group_hbm, group_v)

        srcs = (src_b0, src_b1)
        dsts = (dst_b0, dst_b1)
        ews = (ew_b0, ew_b1)
        cuts = (cut_b0, cut_b1)
        exs = (ex_b0, ex_b1)
        evs = (ev_b0, ev_b1)
        in_sems = (in_sem0, in_sem1)
        out_sems = (out_sem0, out_sem1)

        lane_iota = lax.iota(jnp.int32, _LANES)

        def start_in(k, b):
            base = base_w + k * _CHUNK
            return (
                pltpu.async_copy(ei_hbm.at[pl.ds(base, _CHUNK)],
                                 srcs[b], in_sems[b]),
                pltpu.async_copy(ei_hbm.at[pl.ds(n_edges + base, _CHUNK)],
                                 dsts[b], in_sems[b]),
            )

        def start_out(k, b):
            base = base_w + k * _CHUNK
            row = wid * n_chunks + k
            return (
                pltpu.async_copy(ews[b], ew_hbm.at[pl.ds(base, _CHUNK)],
                                 out_sems[b]),
                pltpu.async_copy(cuts[b], cut_hbm.at[row, 0], out_sems[b]),
                pltpu.async_copy(exs[b], ex_hbm.at[row, 0], out_sems[b]),
                pltpu.async_copy(evs[b],
                                 ev_hbm.at[pl.ds(base * 3, _CHUNK * 3)],
                                 out_sems[b]),
            )

        def compute(b):
            src_v, dst_v = srcs[b], dsts[b]
            ew_v, cut_v, ex_v, ev_v = ews[b], cuts[b], exs[b], evs[b]

            def vec_body(j, carry2):
                off = j * _LANES
                si = src_v[pl.ds(off, _LANES)] * 3
                di = dst_v[pl.ds(off, _LANES)] * 3
                dx = (plsc.load_gather(node_v, [si])
                      - plsc.load_gather(group_v, [di]))
                dy = (plsc.load_gather(node_v, [si + 1])
                      - plsc.load_gather(group_v, [di + 1]))
                dz = (plsc.load_gather(node_v, [si + 2])
                      - plsc.load_gather(group_v, [di + 2]))
                d2 = dx * dx + dy * dy + dz * dz
                bits = plsc.bitcast(d2, jnp.int32)
                bits = 0x5F3759DF - lax.shift_right_logical(bits, 1)
                y = plsc.bitcast(bits, jnp.float32)
                y = y * (1.5 - 0.5 * d2 * y * y)
                y = y * (1.5 - 0.5 * d2 * y * y)
                y = y * (1.5 - 0.5 * d2 * y * y)
                d = d2 * y
                ew_v[pl.ds(off, _LANES)] = d
                u = u_scale * jnp.minimum(d2, cut2)
                cosv = jnp.full((_LANES,), _COS_COEFFS[6], jnp.float32)
                cosv = cosv * u + _COS_COEFFS[5]
                cosv = cosv * u + _COS_COEFFS[4]
                cosv = cosv * u + _COS_COEFFS[3]
                cosv = cosv * u + _COS_COEFFS[2]
                cosv = cosv * u + _COS_COEFFS[1]
                cosv = cosv * u + _COS_COEFFS[0]
                cut = jnp.where(d2 < cut2, 0.5 * (cosv + 1.0), 0.0)
                cut_v[pl.ds(off, _LANES)] = cut
                ex_v[pl.ds(off, _LANES)] = jnp.exp(-alpha * d)
                rows = off * 3 + lane_iota * 3
                plsc.store_scatter(ev_v, [rows], dx * y)
                plsc.store_scatter(ev_v, [rows + 1], dy * y)
                plsc.store_scatter(ev_v, [rows + 2], dz * y)
                return carry2

            lax.fori_loop(0, _CHUNK // _LANES, vec_body, 0)

        in_ds = [None] * n_chunks
        out_ds = [None] * n_chunks
        in_ds[0] = start_in(0, 0)
        for k in range(n_chunks):
            b = k & 1
            if k + 1 < n_chunks:
                in_ds[k + 1] = start_in(k + 1, 1 - b)
            for dsc in in_ds[k]:
                dsc.wait()
            if k >= 2:
                for dsc in out_ds[k - 2]:
                    dsc.wait()
            compute(b)
            out_ds[k] = start_out(k, b)
        for k in range(max(0, n_chunks - 2), n_chunks):
            for dsc in out_ds[k]:
                dsc.wait()

    slot = [
        pltpu.VMEM((_CHUNK,), jnp.int32),
        pltpu.VMEM((_CHUNK,), jnp.int32),
        pltpu.VMEM((_CHUNK,), jnp.float32),
        pltpu.VMEM((_CHUNK,), jnp.float32),
        pltpu.VMEM((_CHUNK,), jnp.float32),
        pltpu.VMEM((_CHUNK * 3,), jnp.float32),
    ]
    k = pl.kernel(
        body,
        out_type=(
            jax.ShapeDtypeStruct((n_edges,), jnp.float32),
            jax.ShapeDtypeStruct((n_rows, 1, _CHUNK), jnp.float32),
            jax.ShapeDtypeStruct((n_rows, 1, _CHUNK), jnp.float32),
            jax.ShapeDtypeStruct((n_edges * 3,), jnp.float32),
        ),
        mesh=mesh,
        compiler_params=pltpu.CompilerParams(needs_layout_passes=False),
        scratch_types=[
            pltpu.VMEM(node_flat.shape, jnp.float32),
            pltpu.VMEM(group_flat.shape, jnp.float32),
        ] + slot + [
            pltpu.VMEM((_CHUNK,), jnp.int32),
            pltpu.VMEM((_CHUNK,), jnp.int32),
            pltpu.VMEM((_CHUNK,), jnp.float32),
            pltpu.VMEM((_CHUNK,), jnp.float32),
            pltpu.VMEM((_CHUNK,), jnp.float32),
            pltpu.VMEM((_CHUNK * 3,), jnp.float32),
            pltpu.SemaphoreType.DMA,
            pltpu.SemaphoreType.DMA,
            pltpu.SemaphoreType.DMA,
            pltpu.SemaphoreType.DMA,
        ],
    )
    return k(edge_index, node_flat, group_flat)


_TC_BLK = _CHUNK


_TC_ROWS = 4


def _tc_rbf_body(cut_ref, ex_ref, means_ref, betas_ref, w_ref, b_ref,
                 attr_ref):
    for j in range(_TC_ROWS):
        ex_row = ex_ref[j].reshape(1, ex_ref.shape[-1])
        cut_row = cut_ref[j].reshape(1, cut_ref.shape[-1])
        t = ex_row - means_ref[...]
        rbf = cut_row * jnp.exp(-betas_ref[...] * t * t)
        attr_ref[pl.ds(j * _TC_BLK, _TC_BLK), :] = lax.dot_general(
            rbf, w_ref[...],
            dimension_numbers=(((0,), (0,)), ((), ())),
            preferred_element_type=jnp.float32,
        ) + b_ref[...]


def _tc_rbf_linear(cut, ex, means, betas, W, b):
    n_edges = cut.shape[0] * cut.shape[2]
    num_rbf, hidden = W.shape
    grid = (n_edges // (_TC_BLK * _TC_ROWS),)
    return pl.pallas_call(
        _tc_rbf_body,
        grid=grid,
        in_specs=[
            pl.BlockSpec((_TC_ROWS, 1, _TC_BLK), lambda i: (i, 0, 0)),
            pl.BlockSpec((_TC_ROWS, 1, _TC_BLK), lambda i: (i, 0, 0)),
            pl.BlockSpec((num_rbf, 1), lambda i: (0, 0)),
            pl.BlockSpec((num_rbf, 1), lambda i: (0, 0)),
            pl.BlockSpec((num_rbf, hidden), lambda i: (0, 0)),
            pl.BlockSpec((1, hidden), lambda i: (0, 0)),
        ],
        out_specs=pl.BlockSpec((_TC_BLK * _TC_ROWS, hidden),
                               lambda i: (i, 0)),
        out_shape=jax.ShapeDtypeStruct((n_edges, hidden), jnp.float32),
    )(
        cut,
        ex,
        means.reshape(num_rbf, 1),
        betas.reshape(num_rbf, 1),
        W,
        b.reshape(1, hidden),
    )


def kernel(edge_index, node_pos, group_pos, means, betas, W, b):
    edge_weight, cut, ex, ev_flat = _sc_gather_kernel(
        edge_index.reshape(-1), node_pos.reshape(-1), group_pos.reshape(-1))
    edge_vec = ev_flat.reshape(-1, 3)
    edge_attr = _tc_rbf_linear(cut, ex, means, betas, W, b)
    return (edge_index, edge_weight, edge_attr, edge_vec)


# speedup vs baseline: 5.2192x; 1.0216x over previous
import functools
import math

import jax
import jax.numpy as jnp
from jax import lax
from jax.experimental import pallas as pl
from jax.experimental.pallas import tpu as pltpu
from jax.experimental.pallas import tpu_sc as plsc

CUTOFF_LOWER = 0.0
CUTOFF_UPPER = 10.0

_NUM_CORES = 2
_NUM_SUBCORES = 16
_NW = _NUM_CORES * _NUM_SUBCORES
_LANES = 16
_CHUNK = 2000


_COS_COEFFS = (1.0, -0.49999985098838806, 0.041666463017463684,
               -0.0013887732056900859, 2.4769053197815083e-05,
               -2.707544979330123e-07, 1.7243751981865785e-09)


def _sc_gather_kernel(edge_index, node_flat, group_flat):
    n_edges = edge_index.shape[0] // 2
    edges_per_w = n_edges // _NW
    n_chunks = edges_per_w // _CHUNK
    n_rows = n_edges // _CHUNK
    alpha = 5.0 / (CUTOFF_UPPER - CUTOFF_LOWER)
    u_scale = (math.pi / CUTOFF_UPPER) ** 2
    cut2 = CUTOFF_UPPER * CUTOFF_UPPER

    mesh = plsc.VectorSubcoreMesh(core_axis_name="c", subcore_axis_name="s")

    def body(ei_hbm, node_hbm, group_hbm,
             ew_hbm, cut_hbm, ex_hbm, ev_hbm,
             node_v, group_v,
             src_b0, dst_b0, ew_b0, cut_b0, ex_b0, ev_b0,
             src_b1, dst_b1, ew_b1, cut_b1, ex_b1, ev_b1,
             in_sem0, in_sem1, out_sem0, out_sem1):
        cid = lax.axis_index("c")
        sid = lax.axis_index("s")
        wid = sid * _NUM_CORES + cid
        base_w = wid * edges_per_w
        pltpu.sync_copy(node_hbm, node_v)
        pltpu.sync_copy(group_hbm, group_v)

        srcs = (src_b0, src_b1)
        dsts = (dst_b0, dst_b1)
        ews = (ew_b0, ew_b1)
        cuts = (cut_b0, cut_b1)
        exs = (ex_b0, ex_b1)
        evs = (ev_b0, ev_b1)
        in_sems = (in_sem0, in_sem1)
        out_sems = (out_sem0, out_sem1)

        lane_iota = lax.iota(jnp.int32, _LANES)

        def start_in(k, b):
            base = base_w + k * _CHUNK
            return (
                pltpu.async_copy(ei_hbm.at[pl.ds(base, _CHUNK)],
                                 srcs[b], in_sems[b]),
                pltpu.async_copy(ei_hbm.at[pl.ds(n_edges + base, _CHUNK)],
                                 dsts[b], in_sems[b]),
            )

        def start_out(k, b):
            base = base_w + k * _CHUNK
            row = wid * n_chunks + k
            return (
                pltpu.async_copy(ews[b], ew_hbm.at[pl.ds(base, _CHUNK)],
                                 out_sems[b]),
                pltpu.async_copy(cuts[b], cut_hbm.at[row, 0], out_sems[b]),
                pltpu.async_copy(exs[b], ex_hbm.at[row, 0], out_sems[b]),
                pltpu.async_copy(evs[b],
                                 ev_hbm.at[pl.ds(base * 3, _CHUNK * 3)],
                                 out_sems[b]),
            )

        def compute(b):
            src_v, dst_v = srcs[b], dsts[b]
            ew_v, cut_v, ex_v, ev_v = ews[b], cuts[b], exs[b], evs[b]

            def vec_body(j, carry2):
                off = j * _LANES
                si = src_v[pl.ds(off, _LANES)] * 3
                di = dst_v[pl.ds(off, _LANES)] * 3
                dx = (plsc.load_gather(node_v, [si])
                      - plsc.load_gather(group_v, [di]))
                dy = (plsc.load_gather(node_v, [si + 1])
                      - plsc.load_gather(group_v, [di + 1]))
                dz = (plsc.load_gather(node_v, [si + 2])
                      - plsc.load_gather(group_v, [di + 2]))
                d2 = dx * dx + dy * dy + dz * dz
                bits = plsc.bitcast(d2, jnp.int32)
                bits = 0x5F3759DF - lax.shift_right_logical(bits, 1)
                y = plsc.bitcast(bits, jnp.float32)
                y = y * (1.5 - 0.5 * d2 * y * y)
                y = y * (1.5 - 0.5 * d2 * y * y)
                y = y * (1.5 - 0.5 * d2 * y * y)
                d = d2 * y
                ew_v[pl.ds(off, _LANES)] = d
                u = u_scale * jnp.minimum(d2, cut2)
                cosv = jnp.full((_LANES,), _COS_COEFFS[6], jnp.float32)
                cosv = cosv * u + _COS_COEFFS[5]
                cosv = cosv * u + _COS_COEFFS[4]
                cosv = cosv * u + _COS_COEFFS[3]
                cosv = cosv * u + _COS_COEFFS[2]
                cosv = cosv * u + _COS_COEFFS[1]
                cosv = cosv * u + _COS_COEFFS[0]
                cut = jnp.where(d2 < cut2, 0.5 * (cosv + 1.0), 0.0)
                cut_v[pl.ds(off, _LANES)] = cut
                ex_v[pl.ds(off, _LANES)] = jnp.exp(-alpha * d)
                rows = off * 3 + lane_iota * 3
                plsc.store_scatter(ev_v, [rows], dx * y)
                plsc.store_scatter(ev_v, [rows + 1], dy * y)
                plsc.store_scatter(ev_v, [rows + 2], dz * y)
                return carry2

            lax.fori_loop(0, _CHUNK // _LANES, vec_body, 0)

        in_ds = [None] * n_chunks
        out_ds = [None] * n_chunks
        in_ds[0] = start_in(0, 0)
        for k in range(n_chunks):
            b = k & 1
            if k + 1 < n_chunks:
                in_ds[k + 1] = start_in(k + 1, 1 - b)
            for dsc in in_ds[k]:
                dsc.wait()
            if k >= 2:
                for dsc in out_ds[k - 2]:
                    dsc.wait()
            compute(b)
            out_ds[k] = start_out(k, b)
        for k in range(max(0, n_chunks - 2), n_chunks):
            for dsc in out_ds[k]:
                dsc.wait()

    slot = [
        pltpu.VMEM((_CHUNK,), jnp.int32),
        pltpu.VMEM((_CHUNK,), jnp.int32),
        pltpu.VMEM((_CHUNK,), jnp.float32),
        pltpu.VMEM((_CHUNK,), jnp.float32),
        pltpu.VMEM((_CHUNK,), jnp.float32),
        pltpu.VMEM((_CHUNK * 3,), jnp.float32),
    ]
    k = pl.kernel(
        body,
        out_type=(
            jax.ShapeDtypeStruct((n_edges,), jnp.float32),
            jax.ShapeDtypeStruct((n_rows, 1, _CHUNK), jnp.float32),
            jax.ShapeDtypeStruct((n_rows, 1, _CHUNK), jnp.float32),
            jax.ShapeDtypeStruct((n_edges * 3,), jnp.float32),
        ),
        mesh=mesh,
        compiler_params=pltpu.CompilerParams(needs_layout_passes=False),
        scratch_types=[
            pltpu.VMEM(node_flat.shape, jnp.float32),
            pltpu.VMEM(group_flat.shape, jnp.float32),
        ] + slot + [
            pltpu.VMEM((_CHUNK,), jnp.int32),
            pltpu.VMEM((_CHUNK,), jnp.int32),
            pltpu.VMEM((_CHUNK,), jnp.float32),
            pltpu.VMEM((_CHUNK,), jnp.float32),
            pltpu.VMEM((_CHUNK,), jnp.float32),
            pltpu.VMEM((_CHUNK * 3,), jnp.float32),
            pltpu.SemaphoreType.DMA,
            pltpu.SemaphoreType.DMA,
            pltpu.SemaphoreType.DMA,
            pltpu.SemaphoreType.DMA,
        ],
    )
    return k(edge_index, node_flat, group_flat)


_TC_BLK = _CHUNK


_TC_ROWS = 8


def _tc_rbf_body(cut_ref, ex_ref, means_ref, betas_ref, w_ref, b_ref,
                 attr_ref):
    for j in range(_TC_ROWS):
        ex_row = ex_ref[j].reshape(1, ex_ref.shape[-1])
        cut_row = cut_ref[j].reshape(1, cut_ref.shape[-1])
        t = ex_row - means_ref[...]
        rbf = cut_row * jnp.exp(-betas_ref[...] * t * t)
        attr_ref[pl.ds(j * _TC_BLK, _TC_BLK), :] = lax.dot_general(
            rbf, w_ref[...],
            dimension_numbers=(((0,), (0,)), ((), ())),
            preferred_element_type=jnp.float32,
        ) + b_ref[...]


def _tc_rbf_linear(cut, ex, means, betas, W, b):
    n_edges = cut.shape[0] * cut.shape[2]
    num_rbf, hidden = W.shape
    grid = (n_edges // (_TC_BLK * _TC_ROWS),)
    return pl.pallas_call(
        _tc_rbf_body,
        grid=grid,
        in_specs=[
            pl.BlockSpec((_TC_ROWS, 1, _TC_BLK), lambda i: (i, 0, 0)),
            pl.BlockSpec((_TC_ROWS, 1, _TC_BLK), lambda i: (i, 0, 0)),
            pl.BlockSpec((num_rbf, 1), lambda i: (0, 0)),
            pl.BlockSpec((num_rbf, 1), lambda i: (0, 0)),
            pl.BlockSpec((num_rbf, hidden), lambda i: (0, 0)),
            pl.BlockSpec((1, hidden), lambda i: (0, 0)),
        ],
        out_specs=pl.BlockSpec((_TC_BLK * _TC_ROWS, hidden),
                               lambda i: (i, 0)),
        out_shape=jax.ShapeDtypeStruct((n_edges, hidden), jnp.float32),
    )(
        cut,
        ex,
        means.reshape(num_rbf, 1),
        betas.reshape(num_rbf, 1),
        W,
        b.reshape(1, hidden),
    )


def kernel(edge_index, node_pos, group_pos, means, betas, W, b):
    edge_weight, cut, ex, ev_flat = _sc_gather_kernel(
        edge_index.reshape(-1), node_pos.reshape(-1), group_pos.reshape(-1))
    edge_vec = ev_flat.reshape(-1, 3)
    edge_attr = _tc_rbf_linear(cut, ex, means, betas, W, b)
    return (edge_index, edge_weight, edge_attr, edge_vec)
